# baseline (speedup 1.0000x reference)
import dataclasses
import functools

import jax
import jax.numpy as jnp
from jax import lax
from jax.experimental import pallas as pl
from jax.experimental.pallas import tpu as pltpu
from jax.experimental.pallas import tpu_sc as plsc

N_NODES = 10000
N_EDGES = 320000
NUM_CORES = 2
NUM_SUBCORES = 16
NW = NUM_CORES * NUM_SUBCORES
PERW = N_EDGES // NW
CHUNK = 128
NCHW = 80
NGRP = NCHW // 4
E_PAD = NW * NCHW * CHUNK
RPW = 632
N_PAD = RPW * NUM_SUBCORES

_MESH = plsc.VectorSubcoreMesh(core_axis_name="c", subcore_axis_name="s")
_SC_PARAMS = pltpu.CompilerParams()
if "needs_layout_passes" in pltpu.CompilerParams.__dataclass_fields__:
    _SC_PARAMS = dataclasses.replace(_SC_PARAMS, needs_layout_passes=False)


def _sc_degree(dst):
    @functools.partial(
        pl.kernel,
        out_type=jax.ShapeDtypeStruct((NW, N_PAD), jnp.float32),
        mesh=_MESH,
        compiler_params=_SC_PARAMS,
        scratch_types=[
            pltpu.VMEM((PERW,), jnp.int32),
            pltpu.VMEM((N_PAD,), jnp.float32),
        ],
    )
    def k(dst_hbm, out_hbm, dst_v, hist_v):
        cid = lax.axis_index("c")
        sid = lax.axis_index("s")
        wid = cid * NUM_SUBCORES + sid

        @pl.loop(0, N_PAD // 16)
        def _(i):
            hist_v[pl.ds(i * 16, 16)] = jnp.zeros((16,), jnp.float32)

        ebase = cid * (N_EDGES // 2) + sid * PERW
        pltpu.sync_copy(dst_hbm.at[pl.ds(ebase, PERW)], dst_v)
        ones = jnp.ones((16,), jnp.float32)

        @pl.loop(0, PERW // 16)
        def _(i):
            idx = dst_v[pl.ds(i * 16, 16)]
            plsc.addupdate_scatter(hist_v, [idx], ones)

        pltpu.sync_copy(hist_v, out_hbm.at[wid])

    return k(dst)


def _tc_degsum(degp):
    nw, npad = degp.shape

    def body(p_ref, dis_ref):
        deg = jnp.sum(p_ref[...], axis=0, keepdims=True) + 1.0
        dis_ref[...] = lax.rsqrt(deg)

    return pl.pallas_call(
        body,
        grid=(1,),
        in_specs=[pl.BlockSpec((nw, npad), lambda i: (0, 0))],
        out_specs=pl.BlockSpec((1, npad), lambda i: (0, 0)),
        out_shape=jax.ShapeDtypeStruct((1, npad), jnp.float32),
    )(degp)


def _make_sc_aggregate(D):
    @functools.partial(
        pl.kernel,
        out_type=jax.ShapeDtypeStruct((NUM_CORES, N_PAD, D), jnp.float32),
        mesh=_MESH,
        scratch_types=[
            pltpu.VMEM((2, CHUNK), jnp.int32),
            pltpu.VMEM((CHUNK, D), jnp.float32),
            pltpu.VMEM_SHARED((N_PAD, D), jnp.float32),
        ],
    )
    def k(y_hbm, sd_hbm, zeros_hbm, out_hbm, sd_v, rows_v, acc_sh):
        cid = lax.axis_index("c")
        sid = lax.axis_index("s")
        wid = cid * NUM_SUBCORES + sid
        pltpu.sync_copy(zeros_hbm.at[pl.ds(sid * RPW, RPW)],
                        acc_sh.at[pl.ds(sid * RPW, RPW)])
        plsc.subcore_barrier()
        cbase = wid * NCHW

        @pl.loop(0, NCHW)
        def _(c):
            pltpu.sync_copy(sd_hbm.at[cbase + c], sd_v)
            pltpu.sync_copy(y_hbm.at[sd_v.at[0]], rows_v)
            pltpu.sync_copy(rows_v, acc_sh.at[sd_v.at[1]], add=True)

        plsc.subcore_barrier()
        pltpu.sync_copy(acc_sh.at[pl.ds(sid * RPW, RPW)],
                        out_hbm.at[cid].at[pl.ds(sid * RPW, RPW)])

    return k


_sc_aggregate_128 = _make_sc_aggregate(128)

_BN = 1000


def _tc_matmul(x, W):
    n, kdim = x.shape
    h = W.shape[1]

    def body(x_ref, w_ref, o_ref):
        o_ref[...] = lax.dot_general(
            x_ref[...], w_ref[...], (((1,), (0,)), ((), ())),
            preferred_element_type=jnp.float32,
            precision=lax.Precision.HIGHEST)

    return pl.pallas_call(
        body,
        grid=(n // _BN,),
        in_specs=[pl.BlockSpec((_BN, kdim), lambda i: (i, 0)),
                  pl.BlockSpec((kdim, h), lambda i: (0, 0))],
        out_specs=pl.BlockSpec((_BN, h), lambda i: (i, 0)),
        out_shape=jax.ShapeDtypeStruct((n, h), jnp.float32),
    )(x, W)


def _tc_dis_scale(dis, xw):
    n, h = xw.shape

    def body(dis_ref, xw_ref, y_ref):
        y_ref[...] = xw_ref[...] * dis_ref[...]

    return pl.pallas_call(
        body,
        grid=(n // _BN,),
        in_specs=[pl.BlockSpec((_BN, 1), lambda i: (i, 0)),
                  pl.BlockSpec((_BN, h), lambda i: (i, 0))],
        out_specs=pl.BlockSpec((_BN, h), lambda i: (i, 0)),
        out_shape=jax.ShapeDtypeStruct((n, h), jnp.float32),
    )(dis, xw)


def _tc_layer2_fuse(a0, a1, y1, dis, b1, W2):
    n, h1 = y1.shape
    h2 = W2.shape[1]

    def body(a0_ref, a1_ref, y1_ref, dis_ref, b1_ref, w2_ref, y2_ref):
        dis = dis_ref[...]
        hidden = dis * (a0_ref[...] + a1_ref[...] + y1_ref[...]) + b1_ref[...]
        hidden = jnp.maximum(hidden, 0.0)
        prod = dis * lax.dot_general(
            hidden, w2_ref[...], (((1,), (0,)), ((), ())),
            preferred_element_type=jnp.float32,
            precision=lax.Precision.HIGHEST)
        y2_ref[...] = jnp.concatenate(
            [prod, jnp.zeros_like(prod)], axis=1)

    return pl.pallas_call(
        body,
        grid=(n // _BN,),
        in_specs=[pl.BlockSpec((_BN, h1), lambda i: (i, 0)),
                  pl.BlockSpec((_BN, h1), lambda i: (i, 0)),
                  pl.BlockSpec((_BN, h1), lambda i: (i, 0)),
                  pl.BlockSpec((_BN, 1), lambda i: (i, 0)),
                  pl.BlockSpec((1, h1), lambda i: (0, 0)),
                  pl.BlockSpec((h1, h2), lambda i: (0, 0))],
        out_specs=pl.BlockSpec((_BN, 2 * h2), lambda i: (i, 0)),
        out_shape=jax.ShapeDtypeStruct((n, 2 * h2), jnp.float32),
    )(a0, a1, y1, dis, b1.reshape(1, h1), W2)


def _tc_layer2_post(a0, a1, y2, dis, b2):
    n, w = y2.shape
    h = w // 2

    def body(a0_ref, a1_ref, y2_ref, dis_ref, b2_ref, h_ref, ss_ref):
        i = pl.program_id(0)
        s = (a0_ref[...] + a1_ref[...] + y2_ref[...])[:, :h]
        out = dis_ref[...] * s + b2_ref[...]
        h_ref[...] = out

        @pl.when(i == 0)
        def _():
            ss_ref[...] = jnp.zeros_like(ss_ref)

        ss_ref[...] += jnp.sum(out * out, axis=0, keepdims=True)

    return pl.pallas_call(
        body,
        grid=(n // _BN,),
        in_specs=[pl.BlockSpec((_BN, w), lambda i: (i, 0)),
                  pl.BlockSpec((_BN, w), lambda i: (i, 0)),
                  pl.BlockSpec((_BN, w), lambda i: (i, 0)),
                  pl.BlockSpec((_BN, 1), lambda i: (i, 0)),
                  pl.BlockSpec((1, h), lambda i: (0, 0))],
        out_specs=[pl.BlockSpec((_BN, h), lambda i: (i, 0)),
                   pl.BlockSpec((1, h), lambda i: (0, 0))],
        out_shape=[jax.ShapeDtypeStruct((n, h), jnp.float32),
                   jax.ShapeDtypeStruct((1, h), jnp.float32)],
    )(a0, a1, y2, dis, b2.reshape(1, h))


def _tc_colnorm_div(h2, ss):
    n, h = h2.shape

    def body(h_ref, ss_ref, o_ref):
        scale = 1.0 / jnp.maximum(jnp.sqrt(ss_ref[...]), 1e-12)
        o_ref[...] = h_ref[...] * scale

    return pl.pallas_call(
        body,
        grid=(n // _BN,),
        in_specs=[pl.BlockSpec((_BN, h), lambda i: (i, 0)),
                  pl.BlockSpec((1, h), lambda i: (0, 0))],
        out_specs=pl.BlockSpec((_BN, h), lambda i: (i, 0)),
        out_shape=jax.ShapeDtypeStruct((n, h), jnp.float32),
    )(h2, ss)


def kernel(x, edge_index, W1, b1, W2, b2):
    src = edge_index[0]
    dst = edge_index[1]
    npad_e = E_PAD - N_EDGES
    src_p = jnp.concatenate([src, jnp.zeros((npad_e,), jnp.int32)])
    dst_p = jnp.concatenate([dst, jnp.full((npad_e,), N_NODES, jnp.int32)])
    sd2 = jnp.stack([src_p.reshape(E_PAD // CHUNK, CHUNK),
                     dst_p.reshape(E_PAD // CHUNK, CHUNK)], axis=1)
    zeros128 = jnp.zeros((N_PAD, 128), jnp.float32)

    degp = _sc_degree(dst)
    xw1 = _tc_matmul(x, W1)
    dis_row = _tc_degsum(degp)
    dis = dis_row[0, :N_NODES].reshape(N_NODES, 1)
    y1 = _tc_dis_scale(dis, xw1)

    agg1 = _sc_aggregate_128(y1, sd2, zeros128)
    y2 = _tc_layer2_fuse(agg1[0, :N_NODES], agg1[1, :N_NODES], y1, dis, b1, W2)

    agg2 = _sc_aggregate_128(y2, sd2, zeros128)
    h2, ss = _tc_layer2_post(agg2[0, :N_NODES], agg2[1, :N_NODES], y2, dis, b2)
    return _tc_colnorm_div(h2, ss)


# speedup vs baseline: 1.2532x; 1.2532x over previous
import dataclasses
import functools

import jax
import jax.numpy as jnp
from jax import lax
from jax.experimental import pallas as pl
from jax.experimental.pallas import tpu as pltpu
from jax.experimental.pallas import tpu_sc as plsc

N_NODES = 10000
N_EDGES = 320000
NUM_CORES = 2
NUM_SUBCORES = 16
NW = NUM_CORES * NUM_SUBCORES
PERW = N_EDGES // NW
CHUNK = 128
NCHW = 80
NGRP = NCHW // 4
E_PAD = NW * NCHW * CHUNK
RPW = 632
N_PAD = RPW * NUM_SUBCORES

_MESH = plsc.VectorSubcoreMesh(core_axis_name="c", subcore_axis_name="s")
_SC_PARAMS = pltpu.CompilerParams()
if "needs_layout_passes" in pltpu.CompilerParams.__dataclass_fields__:
    _SC_PARAMS = dataclasses.replace(_SC_PARAMS, needs_layout_passes=False)


def _sc_degree(dst):
    @functools.partial(
        pl.kernel,
        out_type=jax.ShapeDtypeStruct((NW, N_PAD), jnp.float32),
        mesh=_MESH,
        compiler_params=_SC_PARAMS,
        scratch_types=[
            pltpu.VMEM((PERW,), jnp.int32),
            pltpu.VMEM((N_PAD,), jnp.float32),
        ],
    )
    def k(dst_hbm, out_hbm, dst_v, hist_v):
        cid = lax.axis_index("c")
        sid = lax.axis_index("s")
        wid = cid * NUM_SUBCORES + sid

        @pl.loop(0, N_PAD // 16)
        def _(i):
            hist_v[pl.ds(i * 16, 16)] = jnp.zeros((16,), jnp.float32)

        ebase = cid * (N_EDGES // 2) + sid * PERW
        pltpu.sync_copy(dst_hbm.at[pl.ds(ebase, PERW)], dst_v)
        ones = jnp.ones((16,), jnp.float32)

        @pl.loop(0, PERW // 16)
        def _(i):
            idx = dst_v[pl.ds(i * 16, 16)]
            plsc.addupdate_scatter(hist_v, [idx], ones)

        pltpu.sync_copy(hist_v, out_hbm.at[wid])

    return k(dst)


def _tc_degsum(degp):
    nw, npad = degp.shape

    def body(p_ref, dis_ref):
        deg = jnp.sum(p_ref[...], axis=0, keepdims=True) + 1.0
        dis_ref[...] = lax.rsqrt(deg)

    return pl.pallas_call(
        body,
        grid=(1,),
        in_specs=[pl.BlockSpec((nw, npad), lambda i: (0, 0))],
        out_specs=pl.BlockSpec((1, npad), lambda i: (0, 0)),
        out_shape=jax.ShapeDtypeStruct((1, npad), jnp.float32),
    )(degp)


def _make_sc_aggregate(D):
    @functools.partial(
        pl.kernel,
        out_type=jax.ShapeDtypeStruct((NUM_CORES, N_PAD, D), jnp.float32),
        mesh=_MESH,
        scratch_types=[
            pltpu.VMEM((CHUNK,), jnp.int32),
            pltpu.VMEM((CHUNK,), jnp.int32),
            pltpu.VMEM((CHUNK,), jnp.int32),
            pltpu.VMEM((CHUNK,), jnp.int32),
            pltpu.VMEM((CHUNK, D), jnp.float32),
            pltpu.VMEM((CHUNK, D), jnp.float32),
            pltpu.VMEM_SHARED((N_PAD, D), jnp.float32),
            pltpu.SemaphoreType.DMA,
            pltpu.SemaphoreType.DMA,
            pltpu.SemaphoreType.DMA,
            pltpu.SemaphoreType.DMA,
            pltpu.SemaphoreType.DMA,
            pltpu.SemaphoreType.DMA,
        ],
    )
    def k(y_hbm, src_hbm, dst_hbm, zeros_hbm, out_hbm,
          s0_v, s1_v, d0_v, d1_v, r0_v, r1_v, acc_sh,
          sg0, sg1, ss0, ss1, sd0, sd1):
        cid = lax.axis_index("c")
        sid = lax.axis_index("s")
        wid = cid * NUM_SUBCORES + sid
        pltpu.sync_copy(zeros_hbm.at[pl.ds(sid * RPW, RPW)],
                        acc_sh.at[pl.ds(sid * RPW, RPW)])
        plsc.subcore_barrier()
        ebase = wid * NCHW * CHUNK
        sidx = [s0_v, s1_v]
        didx = [d0_v, d1_v]
        rows = [r0_v, r1_v]
        semg = [sg0, sg1]
        sems = [ss0, ss1]
        semd = [sd0, sd1]

        def issue_idx(c, b):
            e0 = ebase + c * CHUNK
            pltpu.async_copy(src_hbm.at[pl.ds(e0, CHUNK)], sidx[b], sems[b])
            pltpu.async_copy(dst_hbm.at[pl.ds(e0, CHUNK)], didx[b], semd[b])

        def wait_src(b):
            pltpu.make_async_copy(src_hbm.at[pl.ds(0, CHUNK)], sidx[b],
                                  sems[b]).wait()

        def wait_dst(b):
            pltpu.make_async_copy(dst_hbm.at[pl.ds(0, CHUNK)], didx[b],
                                  semd[b]).wait()

        def issue_gather(b):
            pltpu.async_copy(y_hbm.at[sidx[b]], rows[b], semg[b])

        def wait_gather(b):
            pltpu.make_async_copy(y_hbm.at[sidx[b]], rows[b], semg[b]).wait()

        def scatter(b):
            pltpu.sync_copy(rows[b], acc_sh.at[didx[b]], add=True)

        def do_slot(c, b, gather_next=True, idx_next=True):
            if gather_next:
                wait_src(b ^ 1)
                issue_gather(b ^ 1)
            wait_gather(b)
            wait_dst(b)
            scatter(b)
            if idx_next:
                issue_idx(c + 2, b)

        issue_idx(0, 0)
        issue_idx(1, 1)
        wait_src(0)
        issue_gather(0)

        @pl.loop(0, (NCHW - 2) // 2)
        def _(j):
            do_slot(2 * j, 0)
            do_slot(2 * j + 1, 1)

        do_slot(NCHW - 2, 0, idx_next=False)
        do_slot(NCHW - 1, 1, gather_next=False, idx_next=False)

        plsc.subcore_barrier()
        pltpu.sync_copy(acc_sh.at[pl.ds(sid * RPW, RPW)],
                        out_hbm.at[cid].at[pl.ds(sid * RPW, RPW)])

    return k


_sc_aggregate_128 = _make_sc_aggregate(128)

_BN = 1000


def _tc_matmul(x, W):
    n, kdim = x.shape
    h = W.shape[1]

    def body(x_ref, w_ref, o_ref):
        o_ref[...] = lax.dot_general(
            x_ref[...], w_ref[...], (((1,), (0,)), ((), ())),
            preferred_element_type=jnp.float32,
            precision=lax.Precision.HIGHEST)

    return pl.pallas_call(
        body,
        grid=(n // _BN,),
        in_specs=[pl.BlockSpec((_BN, kdim), lambda i: (i, 0)),
                  pl.BlockSpec((kdim, h), lambda i: (0, 0))],
        out_specs=pl.BlockSpec((_BN, h), lambda i: (i, 0)),
        out_shape=jax.ShapeDtypeStruct((n, h), jnp.float32),
    )(x, W)


def _tc_dis_scale(dis, xw):
    n, h = xw.shape

    def body(dis_ref, xw_ref, y_ref):
        y_ref[...] = xw_ref[...] * dis_ref[...]

    return pl.pallas_call(
        body,
        grid=(n // _BN,),
        in_specs=[pl.BlockSpec((_BN, 1), lambda i: (i, 0)),
                  pl.BlockSpec((_BN, h), lambda i: (i, 0))],
        out_specs=pl.BlockSpec((_BN, h), lambda i: (i, 0)),
        out_shape=jax.ShapeDtypeStruct((n, h), jnp.float32),
    )(dis, xw)


def _tc_layer2_fuse(a0, a1, y1, dis, b1, W2):
    n, h1 = y1.shape
    h2 = W2.shape[1]

    def body(a0_ref, a1_ref, y1_ref, dis_ref, b1_ref, w2_ref, y2_ref):
        dis = dis_ref[...]
        hidden = dis * (a0_ref[...] + a1_ref[...] + y1_ref[...]) + b1_ref[...]
        hidden = jnp.maximum(hidden, 0.0)
        prod = dis * lax.dot_general(
            hidden, w2_ref[...], (((1,), (0,)), ((), ())),
            preferred_element_type=jnp.float32,
            precision=lax.Precision.HIGHEST)
        y2_ref[...] = jnp.concatenate(
            [prod, jnp.zeros_like(prod)], axis=1)

    return pl.pallas_call(
        body,
        grid=(n // _BN,),
        in_specs=[pl.BlockSpec((_BN, h1), lambda i: (i, 0)),
                  pl.BlockSpec((_BN, h1), lambda i: (i, 0)),
                  pl.BlockSpec((_BN, h1), lambda i: (i, 0)),
                  pl.BlockSpec((_BN, 1), lambda i: (i, 0)),
                  pl.BlockSpec((1, h1), lambda i: (0, 0)),
                  pl.BlockSpec((h1, h2), lambda i: (0, 0))],
        out_specs=pl.BlockSpec((_BN, 2 * h2), lambda i: (i, 0)),
        out_shape=jax.ShapeDtypeStruct((n, 2 * h2), jnp.float32),
    )(a0, a1, y1, dis, b1.reshape(1, h1), W2)


def _tc_layer2_post(a0, a1, y2, dis, b2):
    n, w = y2.shape
    h = w // 2

    def body(a0_ref, a1_ref, y2_ref, dis_ref, b2_ref, h_ref, ss_ref):
        i = pl.program_id(0)
        s = (a0_ref[...] + a1_ref[...] + y2_ref[...])[:, :h]
        out = dis_ref[...] * s + b2_ref[...]
        h_ref[...] = out

        @pl.when(i == 0)
        def _():
            ss_ref[...] = jnp.zeros_like(ss_ref)

        ss_ref[...] += jnp.sum(out * out, axis=0, keepdims=True)

    return pl.pallas_call(
        body,
        grid=(n // _BN,),
        in_specs=[pl.BlockSpec((_BN, w), lambda i: (i, 0)),
                  pl.BlockSpec((_BN, w), lambda i: (i, 0)),
                  pl.BlockSpec((_BN, w), lambda i: (i, 0)),
                  pl.BlockSpec((_BN, 1), lambda i: (i, 0)),
                  pl.BlockSpec((1, h), lambda i: (0, 0))],
        out_specs=[pl.BlockSpec((_BN, h), lambda i: (i, 0)),
                   pl.BlockSpec((1, h), lambda i: (0, 0))],
        out_shape=[jax.ShapeDtypeStruct((n, h), jnp.float32),
                   jax.ShapeDtypeStruct((1, h), jnp.float32)],
    )(a0, a1, y2, dis, b2.reshape(1, h))


def _tc_colnorm_div(h2, ss):
    n, h = h2.shape

    def body(h_ref, ss_ref, o_ref):
        scale = 1.0 / jnp.maximum(jnp.sqrt(ss_ref[...]), 1e-12)
        o_ref[...] = h_ref[...] * scale

    return pl.pallas_call(
        body,
        grid=(n // _BN,),
        in_specs=[pl.BlockSpec((_BN, h), lambda i: (i, 0)),
                  pl.BlockSpec((1, h), lambda i: (0, 0))],
        out_specs=pl.BlockSpec((_BN, h), lambda i: (i, 0)),
        out_shape=jax.ShapeDtypeStruct((n, h), jnp.float32),
    )(h2, ss)


def kernel(x, edge_index, W1, b1, W2, b2):
    src = edge_index[0]
    dst = edge_index[1]
    npad_e = E_PAD - N_EDGES
    src_p = jnp.concatenate([src, jnp.zeros((npad_e,), jnp.int32)])
    dst_p = jnp.concatenate([dst, jnp.full((npad_e,), N_NODES, jnp.int32)])
    zeros128 = jnp.zeros((N_PAD, 128), jnp.float32)

    degp = _sc_degree(dst)
    xw1 = _tc_matmul(x, W1)
    dis_row = _tc_degsum(degp)
    dis = dis_row[0, :N_NODES].reshape(N_NODES, 1)
    y1 = _tc_dis_scale(dis, xw1)

    agg1 = _sc_aggregate_128(y1, src_p, dst_p, zeros128)
    y2 = _tc_layer2_fuse(agg1[0, :N_NODES], agg1[1, :N_NODES], y1, dis, b1, W2)

    agg2 = _sc_aggregate_128(y2, src_p, dst_p, zeros128)
    h2, ss = _tc_layer2_post(agg2[0, :N_NODES], agg2[1, :N_NODES], y2, dis, b2)
    return _tc_colnorm_div(h2, ss)
